# Initial kernel scaffold; baseline (speedup 1.0000x reference)
import jax
import jax.numpy as jnp
from jax.experimental import pallas as pl


def kernel(x, w_pad, b_pad):
    raise NotImplementedError("write your pallas kernel here")


# speedup vs baseline: 1.1129x; 1.1129x over previous
import jax
import jax.numpy as jnp
from jax.experimental import pallas as pl
from jax.experimental.pallas import tpu as pltpu

_OUT_DIM = 3
_IN_DIM = 6
_PACK = 128
_ROW_TILE = 512


def _packed_mlp_kernel(x_ref, w_ref, b_ref, o_ref):
    x = x_ref[...].astype(jnp.bfloat16)
    h = jnp.dot(x, w_ref[...], preferred_element_type=jnp.float32) + b_ref[...]
    o_ref[...] = 1.0 / (1.0 + jnp.exp(-h))


def kernel(x, w_pad, b_pad):
    B = x.shape[0]
    w = w_pad[:_IN_DIM, :_OUT_DIM]
    b = b_pad[:1, :_OUT_DIM]

    w_big = jnp.kron(jnp.eye(_PACK, dtype=jnp.float32), w).astype(jnp.bfloat16)
    b_big = jnp.tile(b, (1, _PACK))

    rows_per_tile = _PACK * _ROW_TILE
    B_pad = pl.cdiv(B, rows_per_tile) * rows_per_tile
    if B_pad != B:
        x = jnp.pad(x, ((0, B_pad - B), (0, 0)))
    R = B_pad // _PACK
    xr = x.reshape(R, _IN_DIM * _PACK)

    out = pl.pallas_call(
        _packed_mlp_kernel,
        out_shape=jax.ShapeDtypeStruct((R, _OUT_DIM * _PACK), jnp.float32),
        grid=(R // _ROW_TILE,),
        in_specs=[
            pl.BlockSpec((_ROW_TILE, _IN_DIM * _PACK), lambda i: (i, 0)),
            pl.BlockSpec((_IN_DIM * _PACK, _OUT_DIM * _PACK), lambda i: (0, 0)),
            pl.BlockSpec((1, _OUT_DIM * _PACK), lambda i: (0, 0)),
        ],
        out_specs=pl.BlockSpec((_ROW_TILE, _OUT_DIM * _PACK), lambda i: (i, 0)),
        compiler_params=pltpu.CompilerParams(
            dimension_semantics=("parallel",),
        ),
    )(xr, w_big, b_big)

    return out.reshape(B_pad, _OUT_DIM)[:B]


# speedup vs baseline: 46.3754x; 41.6696x over previous
import jax
import jax.numpy as jnp
from jax.experimental import pallas as pl
from jax.experimental.pallas import tpu as pltpu

_OUT_DIM = 3
_IN_DIM = 6
_LANE_TILE = 65536


def _mlp_t_kernel(x_ref, wt_ref, b_ref, o_ref):
    h = jnp.dot(wt_ref[...], x_ref[...],
                preferred_element_type=jnp.float32) + b_ref[...]
    o_ref[...] = 1.0 / (1.0 + jnp.exp(-h))


def kernel(x, w_pad, b_pad):
    B = x.shape[0]
    wt = w_pad[:_IN_DIM, :_OUT_DIM].T
    bt = b_pad[:1, :_OUT_DIM].T
    xt = x.T

    B_pad = pl.cdiv(B, _LANE_TILE) * _LANE_TILE
    if B_pad != B:
        xt = jnp.pad(xt, ((0, 0), (0, B_pad - B)))

    out_t = pl.pallas_call(
        _mlp_t_kernel,
        out_shape=jax.ShapeDtypeStruct((_OUT_DIM, B_pad), jnp.float32),
        grid=(B_pad // _LANE_TILE,),
        in_specs=[
            pl.BlockSpec((_IN_DIM, _LANE_TILE), lambda i: (0, i)),
            pl.BlockSpec((_OUT_DIM, _IN_DIM), lambda i: (0, 0)),
            pl.BlockSpec((_OUT_DIM, 1), lambda i: (0, 0)),
        ],
        out_specs=pl.BlockSpec((_OUT_DIM, _LANE_TILE), lambda i: (0, i)),
        compiler_params=pltpu.CompilerParams(
            dimension_semantics=("parallel",),
        ),
    )(xt, wt, bt)

    return out_t[:, :B].T


# speedup vs baseline: 54.4794x; 1.1747x over previous
import jax
import jax.numpy as jnp
from jax.experimental import pallas as pl
from jax.experimental.pallas import tpu as pltpu

_OUT_DIM = 3
_IN_DIM = 6
_LANE_TILE = 131072


def _mlp_t_kernel(x_ref, wt_ref, b_ref, o_ref):
    h = jnp.dot(wt_ref[...], x_ref[...],
                preferred_element_type=jnp.float32) + b_ref[...]
    o_ref[...] = 1.0 / (1.0 + jnp.exp(-h))


def kernel(x, w_pad, b_pad):
    B = x.shape[0]
    wt = w_pad[:_IN_DIM, :_OUT_DIM].T
    bt = b_pad[:1, :_OUT_DIM].T
    xt = x.T

    B_pad = pl.cdiv(B, _LANE_TILE) * _LANE_TILE
    if B_pad != B:
        xt = jnp.pad(xt, ((0, 0), (0, B_pad - B)))

    out_t = pl.pallas_call(
        _mlp_t_kernel,
        out_shape=jax.ShapeDtypeStruct((_OUT_DIM, B_pad), jnp.float32),
        grid=(B_pad // _LANE_TILE,),
        in_specs=[
            pl.BlockSpec((_IN_DIM, _LANE_TILE), lambda i: (0, i)),
            pl.BlockSpec((_OUT_DIM, _IN_DIM), lambda i: (0, 0)),
            pl.BlockSpec((_OUT_DIM, 1), lambda i: (0, 0)),
        ],
        out_specs=pl.BlockSpec((_OUT_DIM, _LANE_TILE), lambda i: (0, i)),
        compiler_params=pltpu.CompilerParams(
            dimension_semantics=("parallel",),
        ),
    )(xt, wt, bt)

    return out_t[:, :B].T


# speedup vs baseline: 57.6878x; 1.0589x over previous
import jax
import jax.numpy as jnp
from jax.experimental import pallas as pl
from jax.experimental.pallas import tpu as pltpu

_OUT_DIM = 3
_IN_DIM = 6
_LANE_TILE = 262144


def _mlp_t_kernel(x_ref, wt_ref, b_ref, o_ref):
    h = jnp.dot(wt_ref[...], x_ref[...],
                preferred_element_type=jnp.float32) + b_ref[...]
    o_ref[...] = 1.0 / (1.0 + jnp.exp(-h))


def kernel(x, w_pad, b_pad):
    B = x.shape[0]
    wt = w_pad[:_IN_DIM, :_OUT_DIM].T
    bt = b_pad[:1, :_OUT_DIM].T
    xt = x.T

    B_pad = pl.cdiv(B, _LANE_TILE) * _LANE_TILE
    if B_pad != B:
        xt = jnp.pad(xt, ((0, 0), (0, B_pad - B)))

    out_t = pl.pallas_call(
        _mlp_t_kernel,
        out_shape=jax.ShapeDtypeStruct((_OUT_DIM, B_pad), jnp.float32),
        grid=(B_pad // _LANE_TILE,),
        in_specs=[
            pl.BlockSpec((_IN_DIM, _LANE_TILE), lambda i: (0, i)),
            pl.BlockSpec((_OUT_DIM, _IN_DIM), lambda i: (0, 0)),
            pl.BlockSpec((_OUT_DIM, 1), lambda i: (0, 0)),
        ],
        out_specs=pl.BlockSpec((_OUT_DIM, _LANE_TILE), lambda i: (0, i)),
        compiler_params=pltpu.CompilerParams(
            dimension_semantics=("parallel",),
        ),
    )(xt, wt, bt)

    return out_t[:, :B].T


# speedup vs baseline: 59.6123x; 1.0334x over previous
import jax
import jax.numpy as jnp
from jax.experimental import pallas as pl
from jax.experimental.pallas import tpu as pltpu

_OUT_DIM = 3
_IN_DIM = 6
_LANE_TILE = 524288


def _mlp_t_kernel(x_ref, wt_ref, b_ref, o_ref):
    h = jnp.dot(wt_ref[...], x_ref[...],
                preferred_element_type=jnp.float32) + b_ref[...]
    o_ref[...] = 1.0 / (1.0 + jnp.exp(-h))


def kernel(x, w_pad, b_pad):
    B = x.shape[0]
    wt = w_pad[:_IN_DIM, :_OUT_DIM].T
    bt = b_pad[:1, :_OUT_DIM].T
    xt = x.T

    B_pad = pl.cdiv(B, _LANE_TILE) * _LANE_TILE
    if B_pad != B:
        xt = jnp.pad(xt, ((0, 0), (0, B_pad - B)))

    out_t = pl.pallas_call(
        _mlp_t_kernel,
        out_shape=jax.ShapeDtypeStruct((_OUT_DIM, B_pad), jnp.float32),
        grid=(B_pad // _LANE_TILE,),
        in_specs=[
            pl.BlockSpec((_IN_DIM, _LANE_TILE), lambda i: (0, i)),
            pl.BlockSpec((_OUT_DIM, _IN_DIM), lambda i: (0, 0)),
            pl.BlockSpec((_OUT_DIM, 1), lambda i: (0, 0)),
        ],
        out_specs=pl.BlockSpec((_OUT_DIM, _LANE_TILE), lambda i: (0, i)),
        compiler_params=pltpu.CompilerParams(
            dimension_semantics=("parallel",),
        ),
    )(xt, wt, bt)

    return out_t[:, :B].T
